# Initial kernel scaffold; baseline (speedup 1.0000x reference)
import jax
import jax.numpy as jnp
from jax.experimental import pallas as pl


def kernel(x, edge_index, edge_attr, distance_matrix, batch, params):
    raise NotImplementedError("write your pallas kernel here")


# speedup vs baseline: 1.0422x; 1.0422x over previous
import functools

import jax
import jax.numpy as jnp
from jax.experimental import pallas as pl
from jax.experimental.pallas import tpu as pltpu

N = 10000
D = 128
HID = 64
H = 4
C = 64
P = 32
OUTD = 128
NB = 16
MAXD = 100.0
BW = MAXD / NB


_BR = 40


def _dm_stats_kernel(dm_ref, hist_ref, mom_ref):
    x = dm_ref[...]
    xc = x - (MAXD * 0.5)
    s1 = jnp.sum(xc, axis=1)
    s2 = jnp.sum(xc * xc, axis=1)
    mn = jnp.min(x, axis=1)
    mx = jnp.max(x, axis=1)
    cum = [jnp.sum((x < BW * (b + 1)).astype(jnp.float32), axis=1)
           for b in range(NB - 1)]
    cum.append(jnp.full_like(cum[0], float(N)))
    bins = [cum[0]] + [cum[b] - cum[b - 1] for b in range(1, NB)]
    hist_ref[...] = jnp.stack(bins, axis=1)
    mom_ref[...] = jnp.stack(
        [s1, s2, mn, mx, s1, s1, s1, s1], axis=1)


def _dm_stats(dm):
    hist, mom = pl.pallas_call(
        _dm_stats_kernel,
        grid=(N // _BR,),
        in_specs=[pl.BlockSpec((_BR, N), lambda r: (r, 0))],
        out_specs=[
            pl.BlockSpec((_BR, NB), lambda r: (r, 0)),
            pl.BlockSpec((_BR, 8), lambda r: (r, 0)),
        ],
        out_shape=[
            jax.ShapeDtypeStruct((N, NB), jnp.float32),
            jax.ShapeDtypeStruct((N, 8), jnp.float32),
        ],
    )(dm)
    return hist, mom


def _pos_encoding(hist, mom, params):
    hist_n = hist * (1.0 / float(N))
    s1 = mom[:, 0]
    s2 = mom[:, 1]
    mean = s1 * (1.0 / N) + MAXD * 0.5
    var = (s2 - s1 * s1 * (1.0 / N)) * (1.0 / (N - 1))
    std = jnp.sqrt(jnp.maximum(var, 0.0))
    stats = jnp.stack([mean, std, mom[:, 2], mom[:, 3]], axis=1)
    h1 = jax.nn.relu(stats[..., None] * params["deW1"][0][None, None, :]
                     + params["deb1"])
    se = jax.nn.relu(h1 @ params["deW2"][:, :NB] + params["deb2"][:NB])
    se = se.mean(axis=1)
    return jnp.concatenate([hist_n, se], axis=1)


def _conv_layer(xin, src, dst, dattn, cp):
    xpj = (xin @ cp["W"] + cp["b"])
    xpj3 = xpj.reshape(-1, H, C)
    a = cp["attn"][0]
    asn = (xpj3 * a[:, :C]).sum(-1)
    adn = (xpj3 * a[:, C:2 * C]).sum(-1)
    al = asn[src] + adn[dst] + dattn
    al = jax.nn.leaky_relu(al, 0.2)
    al = jnp.exp(al - al.max())
    asum = jax.ops.segment_sum(al, dst, num_segments=xin.shape[0])
    w = al / asum[dst]
    out = jax.ops.segment_sum(xpj3[src] * w[..., None], dst,
                              num_segments=xin.shape[0])
    return out.reshape(-1, H * C)


def kernel(x, edge_index, edge_attr, distance_matrix, batch, params):
    del edge_attr, batch
    dm = distance_matrix
    hist, mom = _dm_stats(dm)
    pos = _pos_encoding(hist, mom, params)

    xp = jax.nn.relu(jnp.concatenate([x, pos], axis=1) @ params["iW"]
                     + params["ib"])
    x_wg = jnp.concatenate([xp, params["gfeat"]], axis=0)

    n = N
    ar = jnp.arange(n, dtype=edge_index.dtype)
    gi = jnp.full((n,), n, dtype=edge_index.dtype)
    src = jnp.concatenate([edge_index[0], gi, ar])
    dst = jnp.concatenate([edge_index[1], ar, gi])

    nmax = n - 1
    ed = dm[jnp.clip(src, 0, nmax), jnp.clip(dst, 0, nmax)][:, None]

    dattns = []
    for lname in ("c1", "c2", "c3", "c4"):
        cp = params[lname]
        ae = cp["attn"][0][:, 2 * C:]
        m2 = cp["dW2"] @ ae.T
        c2v = cp["db2"] @ ae.T
        u = jax.nn.relu(ed * cp["dW1"][0][None, :] + cp["db1"])
        dattns.append(u @ m2 + c2v)

    h1 = jax.nn.elu(_conv_layer(x_wg, src, dst, dattns[0], params["c1"]))
    h2i = h1 + x_wg @ params["s0W"] + params["s0b"]
    h2 = jax.nn.elu(_conv_layer(h2i, src, dst, dattns[1], params["c2"]))
    h3i = h2 + h1 @ params["s1W"] + params["s1b"]
    h3 = jax.nn.elu(_conv_layer(h3i, src, dst, dattns[2], params["c3"]))
    h4i = h3 + h2 @ params["s2W"] + params["s2b"]
    h4 = jax.nn.elu(_conv_layer(h4i, src, dst, dattns[3], params["c4"]))
    return h4[:n] @ params["oW"] + params["ob"]


# speedup vs baseline: 10.9604x; 10.5165x over previous
import functools

import jax
import jax.numpy as jnp
from jax import lax
from jax.experimental import pallas as pl
from jax.experimental.pallas import tpu as pltpu
from jax.experimental.pallas import tpu_sc as plsc

N = 10000
D = 128
HID = 64
H = 4
C = 64
P = 32
OUTD = 128
NB = 16
MAXD = 100.0
BW = MAXD / NB


_BR = 40


def _dm_stats_kernel(dm_ref, hist_ref, mom_ref):
    x = dm_ref[...]
    xc = x - (MAXD * 0.5)
    s1 = jnp.sum(xc, axis=1)
    s2 = jnp.sum(xc * xc, axis=1)
    mn = jnp.min(x, axis=1)
    mx = jnp.max(x, axis=1)
    cum = [jnp.sum((x < BW * (b + 1)).astype(jnp.float32), axis=1)
           for b in range(NB - 1)]
    cum.append(jnp.full_like(cum[0], float(N)))
    bins = [cum[0]] + [cum[b] - cum[b - 1] for b in range(1, NB)]
    hist_ref[...] = jnp.stack(bins, axis=1)
    mom_ref[...] = jnp.stack(
        [s1, s2, mn, mx, s1, s1, s1, s1], axis=1)


def _dm_stats(dm):
    hist, mom = pl.pallas_call(
        _dm_stats_kernel,
        grid=(N // _BR,),
        in_specs=[pl.BlockSpec((_BR, N), lambda r: (r, 0))],
        out_specs=[
            pl.BlockSpec((_BR, NB), lambda r: (r, 0)),
            pl.BlockSpec((_BR, 8), lambda r: (r, 0)),
        ],
        out_shape=[
            jax.ShapeDtypeStruct((N, NB), jnp.float32),
            jax.ShapeDtypeStruct((N, 8), jnp.float32),
        ],
    )(dm)
    return hist, mom


ETP = 180224
NW = 32
EPW = ETP // NW
NGW = EPW // 16
NE_REAL = 160000 + 2 * N
NP = 10240

_mesh = plsc.VectorSubcoreMesh(core_axis_name="c", subcore_axis_name="s")


def _wid():
    return lax.axis_index("s") * 2 + lax.axis_index("c")


def _ed_gather_call(dm128, srcp, dstp):
    @functools.partial(
        pl.kernel, mesh=_mesh,
        out_type=jax.ShapeDtypeStruct((ETP,), jnp.float32),
        scratch_types=[
            pltpu.VMEM((EPW,), jnp.int32),
            pltpu.VMEM((EPW,), jnp.int32),
            pltpu.VMEM((128,), jnp.int32),
            pltpu.VMEM((128,), jnp.int32),
            pltpu.VMEM((128, 128), jnp.float32),
            pltpu.VMEM((EPW,), jnp.float32),
            pltpu.SemaphoreType.DMA,
        ],
    )
    def k(dm_ref, src_ref, dst_ref, out_ref,
          srcb, dstb, rowb, colb, gbuf, edb, sem):
        base = _wid() * EPW
        pltpu.sync_copy(src_ref.at[pl.ds(base, EPW)], srcb)
        pltpu.sync_copy(dst_ref.at[pl.ds(base, EPW)], dstb)
        iot = lax.iota(jnp.int32, 16)

        def body(bq, _):
            off = bq * 128
            for q in range(8):
                sl16 = pl.ds(off + q * 16, 16)
                sv = jnp.minimum(srcb[sl16], N - 1)
                dv = jnp.minimum(dstb[sl16], N - 1)
                qq = sv * N + dv
                rowb[pl.ds(q * 16, 16)] = lax.shift_right_logical(qq, 7)
                colb[pl.ds(q * 16, 16)] = jnp.bitwise_and(qq, 127)
            pltpu.async_copy(dm_ref.at[rowb], gbuf, sem).wait()
            for q in range(8):
                cols = colb[pl.ds(q * 16, 16)]
                clo = jnp.bitwise_and(cols, 15)
                acc = jnp.zeros((16,), jnp.float32)
                for r in range(16):
                    chi = lax.shift_right_logical(cols[r], 4)
                    sub = gbuf[q * 16 + r, pl.ds(chi * 16, 16)]
                    acc = jnp.where(iot == r, sub[clo], acc)
                edb[pl.ds(off + q * 16, 16)] = acc
            return 0

        lax.fori_loop(0, EPW // 128, body, 0)
        pltpu.sync_copy(edb, out_ref.at[pl.ds(base, EPW)])

    return k(dm128, srcp, dstp)


def _s1_call(srcp, dstp, ed, abt, wpack):
    @functools.partial(
        pl.kernel, mesh=_mesh,
        out_type=(jax.ShapeDtypeStruct((H, ETP), jnp.float32),
                  jax.ShapeDtypeStruct((NW, 16), jnp.float32)),
        scratch_types=[
            pltpu.VMEM((EPW,), jnp.int32),
            pltpu.VMEM((EPW,), jnp.int32),
            pltpu.VMEM((EPW,), jnp.float32),
            pltpu.VMEM((128, 128), jnp.float32),
            pltpu.VMEM((128, 128), jnp.float32),
            pltpu.VMEM((H, EPW), jnp.float32),
            pltpu.VMEM((8, 16), jnp.float32),
            pltpu.VMEM((16,), jnp.float32),
            pltpu.SemaphoreType.DMA,
            pltpu.SemaphoreType.DMA,
        ],
    )
    def k(src_ref, dst_ref, ed_ref, ab_ref, wp_ref,
          al_ref, mx_ref,
          srcb, dstb, edb, sg, dg, alb, wpb, mb, sem0, sem1):
        w = _wid()
        base = w * EPW
        pltpu.sync_copy(src_ref.at[pl.ds(base, EPW)], srcb)
        pltpu.sync_copy(dst_ref.at[pl.ds(base, EPW)], dstb)
        pltpu.sync_copy(ed_ref.at[pl.ds(base, EPW)], edb)
        pltpu.sync_copy(wp_ref, wpb)

        w1v = wpb[0]
        b1v = wpb[1]
        w1s = [w1v[p] for p in range(16)]
        b1s = [b1v[p] for p in range(16)]
        m2s = [[wpb[2 + h][p] for p in range(16)] for h in range(H)]
        c2v = wpb[6]
        c2s = [c2v[h] for h in range(H)]
        neg = jnp.float32(-1e30)
        iot = lax.iota(jnp.int32, 16)
        roll4 = jnp.bitwise_and(iot + 4, 15)

        def body(bq, mvec):
            off = bq * 128
            cp0 = pltpu.async_copy(
                ab_ref.at[srcb.at[pl.ds(off, 128)]], sg, sem0)
            cp1 = pltpu.async_copy(
                ab_ref.at[dstb.at[pl.ds(off, 128)]], dg, sem1)
            cp0.wait()
            cp1.wait()
            for q in range(8):
                sl = pl.ds(off + q * 16, 16)
                edv = edb[sl]
                us = [jnp.maximum(edv * w1s[p] + b1s[p], 0.0)
                      for p in range(16)]
                accs = [jnp.zeros((16,), jnp.float32) for _ in range(H)]
                for r in range(16):
                    srow = sg[q * 16 + r, pl.ds(0, 16)]
                    drow = dg[q * 16 + r, pl.ds(0, 16)]
                    comb = srow + drow[roll4]
                    for h in range(H):
                        accs[h] = jnp.where(iot == r, comb[h], accs[h])
                valid = (base + off + q * 16 + iot) < NE_REAL
                for h in range(H):
                    dat = jnp.broadcast_to(c2s[h], (16,))
                    for p in range(16):
                        dat = dat + us[p] * m2s[h][p]
                    al = accs[h] + dat
                    al = jnp.where(al >= 0.0, al, al * 0.2)
                    al = jnp.where(valid, al, neg)
                    alb[h, sl] = al
                    mvec = jnp.maximum(mvec, al)
            return mvec

        mvec = lax.fori_loop(0, EPW // 128, body,
                             jnp.full((16,), neg, jnp.float32))
        mb[...] = mvec
        pltpu.sync_copy(alb, al_ref.at[:, pl.ds(base, EPW)])
        pltpu.sync_copy(mb, mx_ref.at[w])

    return k(srcp, dstp, ed, abt, wpack)


_EPW2 = ETP // 16
_NB2 = _EPW2 // 128
NBASE = N + 1
NPA = 10240


_CH = 8
_NCH = _NB2 // _CH
_CE = _CH * 128


def _s2_call(srcp2, dstp2, al, m16, xpj_halves):
    @functools.partial(
        pl.kernel, mesh=_mesh,
        out_type=jax.ShapeDtypeStruct((2, NPA, 128), jnp.float32),
        scratch_types=[
            pltpu.VMEM((_CH, 128), jnp.int32),
            pltpu.VMEM((_CH, 128), jnp.int32),
            pltpu.VMEM((64,), jnp.int32),
            pltpu.VMEM((64,), jnp.int32),
            pltpu.VMEM((_CE,), jnp.float32),
            pltpu.VMEM((_CE,), jnp.float32),
            pltpu.VMEM((16,), jnp.float32),
            pltpu.VMEM((128, 128), jnp.float32),
            pltpu.VMEM((128, 128), jnp.float32),
            pltpu.VMEM((64, 128), jnp.float32),
            pltpu.VMEM_SHARED((NPA, 128), jnp.float32),
            pltpu.SemaphoreType.DMA,
            pltpu.SemaphoreType.DMA,
        ],
    )
    def k(src_ref, dst_ref, al_ref, mx_ref, xpj_ref,
          out_ref,
          srcb, dstb, d2b, colb, ea0, ea1, mxb, gb0, gb1, sb2, acc,
          sem0, sem1):
        c = lax.axis_index("c")
        s = lax.axis_index("s")
        rbase = s * _NB2
        ebase = s * _EPW2
        pltpu.sync_copy(mx_ref, mxb)
        mv = mxb[...]
        zv = jnp.zeros((16,), jnp.float32)
        iot16 = lax.iota(jnp.int32, 16)

        def zb_body(r, _):
            for j in range(8):
                sb2[r, pl.ds(j * 16, 16)] = zv
            return 0

        lax.fori_loop(0, 64, zb_body, 0)

        for i in range(NPA // 16 // 64):
            pltpu.sync_copy(
                sb2, acc.at[pl.ds(s * (NPA // 16) + i * 64, 64)])
        plsc.subcore_barrier()

        table = xpj_ref.at[c]

        def process(b, gbuf):
            for h in range(2):
                def isub(q, _):
                    dv = dstb[b, pl.ds(h * 64 + q * 16, 16)]
                    d2b[pl.ds(q * 16, 16)] = (
                        NBASE + lax.shift_right_logical(dv, 6))
                    colb[pl.ds(q * 16, 16)] = jnp.bitwise_and(dv, 63) * 2
                    return 0

                lax.fori_loop(0, 4, isub, 0)

                def dsub(q, _):
                    e0 = ea0[pl.ds(b * 128 + h * 64 + q * 16, 16)]
                    e1 = ea1[pl.ds(b * 128 + h * 64 + q * 16, 16)]
                    cols = colb[pl.ds(q * 16, 16)]
                    for e in range(16):
                        r = q * 16 + e
                        col = cols[e]
                        blk = lax.shift_right_logical(col, 4)
                        clo = jnp.bitwise_and(col, 15)
                        vec = jnp.where(iot16 == clo, e0[e],
                                        jnp.where(iot16 == clo + 1,
                                                  e1[e], 0.0))
                        sb2[r, pl.ds(blk * 16, 16)] = vec
                    return 0

                lax.fori_loop(0, 4, dsub, 0)
                pltpu.sync_copy(sb2, acc.at[d2b], add=True)

                def zsub(q, _):
                    cols = colb[pl.ds(q * 16, 16)]
                    for e in range(16):
                        blk = lax.shift_right_logical(cols[e], 4)
                        sb2[q * 16 + e, pl.ds(blk * 16, 16)] = zv
                    return 0

                lax.fori_loop(0, 4, zsub, 0)

            def sub(q, _):
                e0 = ea0[pl.ds(b * 128 + q * 16, 16)]
                e1 = ea1[pl.ds(b * 128 + q * 16, 16)]
                for e in range(16):
                    s0 = e0[e]
                    s1 = e1[e]
                    r = q * 16 + e
                    for j in range(4):
                        gbuf[r, pl.ds(j * 16, 16)] = (
                            gbuf[r, pl.ds(j * 16, 16)] * s0)
                    for j in range(4, 8):
                        gbuf[r, pl.ds(j * 16, 16)] = (
                            gbuf[r, pl.ds(j * 16, 16)] * s1)
                return 0

            lax.fori_loop(0, 8, sub, 0)
            pltpu.sync_copy(gbuf, acc.at[dstb.at[b]], add=True)

        def chunk(ch, _):
            pltpu.sync_copy(src_ref.at[pl.ds(rbase + ch * _CH, _CH)], srcb)
            pltpu.sync_copy(dst_ref.at[pl.ds(rbase + ch * _CH, _CH)], dstb)
            pltpu.sync_copy(
                al_ref.at[2 * c, pl.ds(ebase + ch * _CE, _CE)], ea0)
            pltpu.sync_copy(
                al_ref.at[2 * c + 1, pl.ds(ebase + ch * _CE, _CE)], ea1)

            def exp_body(g, _):
                ea0[pl.ds(g * 16, 16)] = jnp.exp(ea0[pl.ds(g * 16, 16)] - mv)
                ea1[pl.ds(g * 16, 16)] = jnp.exp(ea1[pl.ds(g * 16, 16)] - mv)
                return 0

            lax.fori_loop(0, _CE // 16, exp_body, 0)

            pltpu.async_copy(table.at[srcb.at[0]], gb0, sem0)
            for b in range(_CH):
                gbuf, sem = (gb0, sem0) if b % 2 == 0 else (gb1, sem1)
                nbuf, nsem = (gb1, sem1) if b % 2 == 0 else (gb0, sem0)
                pltpu.make_async_copy(table.at[srcb.at[b]], gbuf, sem).wait()
                if b + 1 < _CH:
                    pltpu.async_copy(table.at[srcb.at[b + 1]], nbuf, nsem)
                process(b, gbuf)
            return 0

        lax.fori_loop(0, _NCH, chunk, 0)
        plsc.subcore_barrier()
        pltpu.sync_copy(acc.at[pl.ds(s * (NPA // 16), NPA // 16)],
                        out_ref.at[c].at[pl.ds(s * (NPA // 16), NPA // 16)])

    return k(srcp2, dstp2, al, m16, xpj_halves)


def _pos_encoding(hist, mom, params):
    hist_n = hist * (1.0 / float(N))
    s1 = mom[:, 0]
    s2 = mom[:, 1]
    mean = s1 * (1.0 / N) + MAXD * 0.5
    var = (s2 - s1 * s1 * (1.0 / N)) * (1.0 / (N - 1))
    std = jnp.sqrt(jnp.maximum(var, 0.0))
    stats = jnp.stack([mean, std, mom[:, 2], mom[:, 3]], axis=1)
    h1 = jax.nn.relu(stats[..., None] * params["deW1"][0][None, None, :]
                     + params["deb1"])
    se = jax.nn.relu(h1 @ params["deW2"][:, :NB] + params["deb2"][:NB])
    se = se.mean(axis=1)
    return jnp.concatenate([hist_n, se], axis=1)


def _mk_wpack(cp):
    ae = cp["attn"][0][:, 2 * C:]
    m2t = (cp["dW2"] @ ae.T).T
    c2v = cp["db2"] @ ae.T
    return jnp.concatenate([
        cp["dW1"][0][None, :],
        cp["db1"][None, :],
        m2t,
        jnp.pad(c2v, (0, 12))[None, :],
        jnp.zeros((1, 16), jnp.float32),
    ], axis=0)


def _conv_sc(xin, ctx, cp, wpack):
    xpj = (xin @ cp["W"] + cp["b"])
    xpj3 = xpj.reshape(-1, H, C)
    a = cp["attn"][0]
    asn = (xpj3 * a[:, :C]).sum(-1)
    adn = (xpj3 * a[:, C:2 * C]).sum(-1)
    abt = (jnp.zeros((NP, 128), jnp.float32)
           .at[:N + 1, 0:H].set(asn)
           .at[:N + 1, H:2 * H].set(adn))
    xpj_p = jnp.zeros((NP, H * C), jnp.float32).at[:N + 1].set(xpj)
    halves = jnp.stack([xpj_p[:, :128], xpj_p[:, 128:]])

    al, mx = _s1_call(ctx["srcp"], ctx["dstp"], ctx["ed"], abt, wpack)
    m16 = jnp.full((16,), jnp.max(mx), jnp.float32)
    numer = _s2_call(ctx["srcp2"], ctx["dstp2"], al, m16, halves)
    out = jnp.concatenate([numer[0, :N + 1, :],
                           numer[1, :N + 1, :]], axis=1)
    dn = numer[:, NBASE:NBASE + 157, :].reshape(2, 157 * 64, 2)[:, :N + 1]
    asum = jnp.concatenate([dn[0], dn[1]], axis=1)
    return out / jnp.repeat(asum, C, axis=1)


def kernel(x, edge_index, edge_attr, distance_matrix, batch, params):
    del edge_attr, batch
    dm = distance_matrix
    hist, mom = _dm_stats(dm)
    pos = _pos_encoding(hist, mom, params)

    xp = jax.nn.relu(jnp.concatenate([x, pos], axis=1) @ params["iW"]
                     + params["ib"])
    x_wg = jnp.concatenate([xp, params["gfeat"]], axis=0)

    n = N
    ar = jnp.arange(n, dtype=jnp.int32)
    gi = jnp.full((n,), n, dtype=jnp.int32)
    padv = jnp.full((ETP - NE_REAL,), n, dtype=jnp.int32)
    srcp = jnp.concatenate([edge_index[0].astype(jnp.int32), gi, ar, padv])
    dstp = jnp.concatenate([edge_index[1].astype(jnp.int32), ar, gi, padv])

    ed = _ed_gather_call(dm.reshape(N * N // 128, 128), srcp, dstp)

    ctx = {
        "srcp": srcp,
        "dstp": dstp,
        "srcp2": srcp.reshape(ETP // 128, 128),
        "dstp2": dstp.reshape(ETP // 128, 128),
        "ed": ed,
    }

    h1 = jax.nn.elu(_conv_sc(x_wg, ctx, params["c1"], _mk_wpack(params["c1"])))
    h2i = h1 + x_wg @ params["s0W"] + params["s0b"]
    h2 = jax.nn.elu(_conv_sc(h2i, ctx, params["c2"], _mk_wpack(params["c2"])))
    h3i = h2 + h1 @ params["s1W"] + params["s1b"]
    h3 = jax.nn.elu(_conv_sc(h3i, ctx, params["c3"], _mk_wpack(params["c3"])))
    h4i = h3 + h2 @ params["s2W"] + params["s2b"]
    h4 = jax.nn.elu(_conv_sc(h4i, ctx, params["c4"], _mk_wpack(params["c4"])))
    return h4[:n] @ params["oW"] + params["ob"]
